# baseline (speedup 1.0000x reference)
import functools

import jax
import jax.numpy as jnp
from jax import lax
from jax.experimental import pallas as pl
from jax.experimental.pallas import tpu as pltpu
from jax.experimental.pallas import tpu_sc as plsc

NF = 32
L = 16
NC = 2
NS = 16
NW = NC * NS
CH = 128


def _body(uc_hbm, ic_hbm, uft_hbm, ift_hbm, params_hbm, out_hbm,
          uidx_v, iidx_v, uvals_v, ivals_v, params_v, out_v, usem, isem,
          *, bpw):
    nchunk = bpw // CH
    wid = lax.axis_index("s") * NC + lax.axis_index("c")
    base = wid * bpw

    for j in range(nchunk):
        pltpu.sync_copy(uc_hbm.at[pl.ds(base + j * CH, CH)], uidx_v.at[j])
        pltpu.sync_copy(ic_hbm.at[pl.ds(base + j * CH, CH)], iidx_v.at[j])
    pltpu.sync_copy(params_hbm, params_v)

    def fgather(f, carry):
        copies = []
        for j in range(nchunk):
            copies.append(pltpu.async_copy(
                uft_hbm.at[f].at[uidx_v.at[j]],
                uvals_v.at[f, pl.ds(j * CH, CH)], usem))
            copies.append(pltpu.async_copy(
                ift_hbm.at[f].at[iidx_v.at[j]],
                ivals_v.at[f, pl.ds(j * CH, CH)], isem))
        for c in copies:
            c.wait()
        return carry

    lax.fori_loop(0, NF, fgather, 0)

    whalves = [params_v[pl.ds(0, L)], params_v[pl.ds(L, L)]]
    bias = jnp.take_along_axis(params_v[pl.ds(NF, L)],
                               jnp.zeros((L,), jnp.int32), axis=0)

    def group(g, carry):
        sl = pl.ds(g * L, L)
        acc = bias
        for f in range(NF):
            w = jnp.take_along_axis(whalves[f // L],
                                    jnp.full((L,), f % L, jnp.int32), axis=0)
            acc = acc + uvals_v[f, sl] * ivals_v[f, sl] * w
        out_v[sl] = 1.0 / (1.0 + jnp.exp(-acc))
        return carry

    lax.fori_loop(0, bpw // L, group, 0)
    pltpu.sync_copy(out_v, out_hbm.at[pl.ds(base, bpw)])


def kernel(user_coordinates, item_coordinates, user_factors, item_factors, W1, b1):
    batch = user_coordinates.shape[0]
    assert batch % (NW * CH) == 0
    bpw = batch // NW

    params = jnp.concatenate([W1.reshape(-1), b1.reshape(-1)])
    params = jnp.pad(params, (0, NF + L - params.shape[0]))

    mesh = plsc.VectorSubcoreMesh(core_axis_name="c", subcore_axis_name="s")
    run = functools.partial(
        pl.kernel,
        mesh=mesh,
        compiler_params=pltpu.CompilerParams(needs_layout_passes=False,
                                             use_tc_tiling_on_sc=False),
        out_type=jax.ShapeDtypeStruct((batch,), jnp.float32),
        scratch_types=[
            pltpu.VMEM((bpw // CH, CH), jnp.int32),
            pltpu.VMEM((bpw // CH, CH), jnp.int32),
            pltpu.VMEM((NF, bpw), jnp.float32),
            pltpu.VMEM((NF, bpw), jnp.float32),
            pltpu.VMEM((NF + L,), jnp.float32),
            pltpu.VMEM((bpw,), jnp.float32),
            pltpu.SemaphoreType.DMA,
            pltpu.SemaphoreType.DMA,
        ],
    )(functools.partial(_body, bpw=bpw))

    out = run(user_coordinates.astype(jnp.int32),
              item_coordinates.astype(jnp.int32),
              user_factors.T, item_factors.T, params)
    return out.reshape(batch, 1)


# speedup vs baseline: 5.5858x; 5.5858x over previous
import functools

import jax
import jax.numpy as jnp
from jax import lax
from jax.experimental import pallas as pl
from jax.experimental.pallas import tpu as pltpu
from jax.experimental.pallas import tpu_sc as plsc

NF = 32
L = 16
NC = 2
NS = 16
NW = NC * NS
CH = 128
QR = 128
NPASS = 2


def _body(uc_hbm, ic_hbm, uq_hbm, iq_hbm, params_hbm, out_hbm,
          uidx_v, iidx_v, uridx_v, iridx_v, urows_v, irows_v,
          params_v, out_v, usem, isem, *, bpw):
    nchunk = bpw // CH
    bpp = bpw // NPASS
    wid = lax.axis_index("s") * NC + lax.axis_index("c")
    base = wid * bpw

    pltpu.sync_copy(uc_hbm.at[pl.ds(base, bpw)], uidx_v)
    pltpu.sync_copy(ic_hbm.at[pl.ds(base, bpw)], iidx_v)
    pltpu.sync_copy(params_hbm, params_v)

    for k in range(bpw // L):
        j, r = k // (CH // L), k % (CH // L)
        sl = pl.ds(k * L, L)
        uridx_v[j, pl.ds(r * L, L)] = uidx_v[sl] >> 2
        iridx_v[j, pl.ds(r * L, L)] = iidx_v[sl] >> 2

    whalves = [params_v[pl.ds(0, L)], params_v[pl.ds(L, L)]]
    bias = jnp.take_along_axis(params_v[pl.ds(NF, L)],
                               jnp.zeros((L,), jnp.int32), axis=0)

    for p in range(NPASS):
        copies = []
        for j in range(nchunk // NPASS):
            c = p * (nchunk // NPASS) + j
            copies.append(pltpu.async_copy(
                uq_hbm.at[uridx_v.at[c]],
                urows_v.at[pl.ds(j * CH, CH)], usem))
            copies.append(pltpu.async_copy(
                iq_hbm.at[iridx_v.at[c]],
                irows_v.at[pl.ds(j * CH, CH)], isem))
        for c in copies:
            c.wait()

        def group(g, carry):
            sl = pl.ds(p * bpp + g * L, L)
            rows = g * L + lax.iota(jnp.int32, L)
            ucol = (uidx_v[sl] & 3) << 5
            icol = (iidx_v[sl] & 3) << 5
            acc = bias
            for f in range(NF):
                pu = plsc.load_gather(urows_v, [rows, ucol + f])
                pi = plsc.load_gather(irows_v, [rows, icol + f])
                w = jnp.take_along_axis(whalves[f // L],
                                        jnp.full((L,), f % L, jnp.int32),
                                        axis=0)
                acc = acc + pu * pi * w
            out_v[sl] = 1.0 / (1.0 + jnp.exp(-acc))
            return carry

        lax.fori_loop(0, bpp // L, group, 0)

    pltpu.sync_copy(out_v, out_hbm.at[pl.ds(base, bpw)])


def kernel(user_coordinates, item_coordinates, user_factors, item_factors, W1, b1):
    batch = user_coordinates.shape[0]
    assert batch % (NW * CH) == 0
    bpw = batch // NW
    nrow = user_factors.shape[0]

    uq = user_factors.reshape(nrow * NF // QR, QR)
    iq = item_factors.reshape(nrow * NF // QR, QR)

    params = jnp.concatenate([W1.reshape(-1), b1.reshape(-1)])
    params = jnp.pad(params, (0, NF + L - params.shape[0]))

    mesh = plsc.VectorSubcoreMesh(core_axis_name="c", subcore_axis_name="s")
    run = functools.partial(
        pl.kernel,
        mesh=mesh,
        compiler_params=pltpu.CompilerParams(needs_layout_passes=False,
                                             use_tc_tiling_on_sc=True),
        out_type=jax.ShapeDtypeStruct((batch,), jnp.float32),
        scratch_types=[
            pltpu.VMEM((bpw,), jnp.int32),
            pltpu.VMEM((bpw,), jnp.int32),
            pltpu.VMEM((bpw // CH, CH), jnp.int32),
            pltpu.VMEM((bpw // CH, CH), jnp.int32),
            pltpu.VMEM((bpw // NPASS, QR), jnp.float32),
            pltpu.VMEM((bpw // NPASS, QR), jnp.float32),
            pltpu.VMEM((NF + L, ), jnp.float32),
            pltpu.VMEM((bpw,), jnp.float32),
            pltpu.SemaphoreType.DMA,
            pltpu.SemaphoreType.DMA,
        ],
    )(functools.partial(_body, bpw=bpw))

    out = run(user_coordinates.astype(jnp.int32),
              item_coordinates.astype(jnp.int32),
              uq, iq, params)
    return out.reshape(batch, 1)
